# Initial kernel scaffold; baseline (speedup 1.0000x reference)
import jax
import jax.numpy as jnp
from jax.experimental import pallas as pl


def kernel(xyz_bcn, feat, Wq, bq, Wk, bk, Wv, bv, Wp, bp, gp, bgp, Wa, ba, ga, bga, gn_g, gn_b):
    raise NotImplementedError("write your pallas kernel here")


# speedup vs baseline: 34.6490x; 34.6490x over previous
import functools
import jax
import jax.numpy as jnp
from jax import lax
from jax.experimental import pallas as pl
from jax.experimental.pallas import tpu as pltpu
from jax.experimental.pallas import tpu_sc as plsc

N = 8192
DIM = 128
K = 16
NCHUNK = 512
CHW = N // NCHUNK
NK = N * K
BIG = 3e38

_MT = 512


def _t0_body(xyzm_ref, xyzl_ref, cm_ref, xx_ref, xr_ref):
    xm = xyzm_ref[...]
    xl = xyzl_ref[...]
    g = jnp.dot(xm, xl, preferred_element_type=jnp.float32)
    xx = jnp.sum(xm * xm, axis=1, keepdims=True)
    xxc = jnp.sum(xl * xl, axis=0, keepdims=True)
    z = jnp.maximum((xx + xxc) - 2.0 * g, 0.0)
    xx_ref[...] = xx
    xr_ref[...] = xm.astype(jnp.bfloat16).astype(jnp.float32)
    i = pl.program_id(0)

    @pl.when(i == 0)
    def _init():
        cm_ref[...] = z

    @pl.when(i > 0)
    def _acc():
        cm_ref[...] = jnp.minimum(cm_ref[...], z)


def _t0(xyzT8, xyz8):
    return pl.pallas_call(
        _t0_body,
        grid=(N // _MT,),
        in_specs=[
            pl.BlockSpec((_MT, 8), lambda i: (i, 0)),
            pl.BlockSpec((8, N), lambda i: (0, 0)),
        ],
        out_specs=[
            pl.BlockSpec((_MT, N), lambda i: (0, 0)),
            pl.BlockSpec((_MT, 1), lambda i: (i, 0)),
            pl.BlockSpec((_MT, 8), lambda i: (i, 0)),
        ],
        out_shape=[
            jax.ShapeDtypeStruct((_MT, N), jnp.float32),
            jax.ShapeDtypeStruct((N, 1), jnp.float32),
            jax.ShapeDtypeStruct((N, 8), jnp.float32),
        ],
    )(xyzT8, xyz8)


_NT = 1024


def _t0b_body(f_ref, a1t_ref, a2t_ref, wvt_ref, bka_ref, bqa_ref, bv_ref,
              ka_ref, qa_ref, v_ref):
    f = f_ref[...]
    ka_ref[...] = jnp.dot(f, a1t_ref[...], preferred_element_type=jnp.float32) + bka_ref[...]
    qa_ref[...] = jnp.dot(f, a2t_ref[...], preferred_element_type=jnp.float32) + bqa_ref[...]
    v_ref[...] = jnp.dot(f, wvt_ref[...], preferred_element_type=jnp.float32) + bv_ref[...]


def _t0b(fT, a1t, a2t, wvt, bka, bqa, bv):
    full = pl.BlockSpec((DIM, DIM), lambda i: (0, 0))
    row = pl.BlockSpec((1, DIM), lambda i: (0, 0))
    blk = pl.BlockSpec((_NT, DIM), lambda i: (i, 0))
    return pl.pallas_call(
        _t0b_body,
        grid=(N // _NT,),
        in_specs=[blk, full, full, full, row, row, row],
        out_specs=[blk, blk, blk],
        out_shape=[jax.ShapeDtypeStruct((N, DIM), jnp.float32)] * 3,
    )(fT, a1t, a2t, wvt, bka, bqa, bv)


_QT = 128


def _t1_body(cm_ref, sel_ref):
    cm = cm_ref[...]
    ci = lax.broadcasted_iota(jnp.int32, (NCHUNK, _QT), 0)
    rows = []
    for _ in range(K):
        m = jnp.min(cm, axis=0, keepdims=True)
        am = jnp.min(jnp.where(cm <= m, ci, NCHUNK), axis=0, keepdims=True)
        rows.append(am)
        cm = jnp.where(ci == am, BIG, cm)
    sel = jnp.concatenate(rows, axis=0)
    sel_ref[...] = sel.T


def _t1(cmt):
    return pl.pallas_call(
        _t1_body,
        grid=(N // _QT,),
        in_specs=[pl.BlockSpec((NCHUNK, _QT), lambda i: (0, i))],
        out_specs=pl.BlockSpec((_QT, K), lambda i: (i, 0)),
        out_shape=jax.ShapeDtypeStruct((N, K), jnp.int32),
    )(cmt)


_PT = 2048


def _t3b_body(delta_ref, wp8_ref, bp_ref, st_ref):
    pe = jnp.dot(delta_ref[...], wp8_ref[...], preferred_element_type=jnp.float32) + bp_ref[...]
    s = jnp.sum(pe, axis=0, keepdims=True)
    ss = jnp.sum(pe * pe, axis=0, keepdims=True)
    blk = jnp.concatenate([s, ss, jnp.zeros((6, DIM), jnp.float32)], axis=0)
    i = pl.program_id(0)

    @pl.when(i == 0)
    def _init():
        st_ref[...] = blk

    @pl.when(i > 0)
    def _acc():
        st_ref[...] = st_ref[...] + blk


def _t3b(delta, wp8, bp):
    return pl.pallas_call(
        _t3b_body,
        grid=(NK // _PT,),
        in_specs=[
            pl.BlockSpec((_PT, 8), lambda i: (i, 0)),
            pl.BlockSpec((8, DIM), lambda i: (0, 0)),
            pl.BlockSpec((1, DIM), lambda i: (0, 0)),
        ],
        out_specs=pl.BlockSpec((8, DIM), lambda i: (0, 0)),
        out_shape=jax.ShapeDtypeStruct((8, DIM), jnp.float32),
    )(delta, wp8, bp)


def _t4_body(kj_ref, qa_ref, delta_ref, wp8_ref, bp_ref, scp_ref, offp_ref,
             wat_ref, spre_ref, st_ref):
    pe_pre = jnp.dot(delta_ref[...], wp8_ref[...], preferred_element_type=jnp.float32) + bp_ref[...]
    pe = jnp.maximum(pe_pre * scp_ref[...] + offp_ref[...], 0.0)
    qa = qa_ref[...]
    qa_rep = jnp.broadcast_to(qa[:, None, :], (_PT // K, K, DIM)).reshape(_PT, DIM)
    spre = kj_ref[...] - qa_rep + jnp.dot(pe, wat_ref[...], preferred_element_type=jnp.float32)
    spre_ref[...] = spre
    s = jnp.sum(spre, axis=0, keepdims=True)
    ss = jnp.sum(spre * spre, axis=0, keepdims=True)
    blk = jnp.concatenate([s, ss, jnp.zeros((6, DIM), jnp.float32)], axis=0)
    i = pl.program_id(0)

    @pl.when(i == 0)
    def _init():
        st_ref[...] = blk

    @pl.when(i > 0)
    def _acc():
        st_ref[...] = st_ref[...] + blk


def _t4(kj, qa, delta, wp8, bp, scp, offp, wat):
    row = pl.BlockSpec((1, DIM), lambda i: (0, 0))
    return pl.pallas_call(
        _t4_body,
        grid=(NK // _PT,),
        in_specs=[
            pl.BlockSpec((_PT, DIM), lambda i: (i, 0)),
            pl.BlockSpec((_PT // K, DIM), lambda i: (i, 0)),
            pl.BlockSpec((_PT, 8), lambda i: (i, 0)),
            pl.BlockSpec((8, DIM), lambda i: (0, 0)),
            row, row, row,
            pl.BlockSpec((DIM, DIM), lambda i: (0, 0)),
        ],
        out_specs=[
            pl.BlockSpec((_PT, DIM), lambda i: (i, 0)),
            pl.BlockSpec((8, DIM), lambda i: (0, 0)),
        ],
        out_shape=[
            jax.ShapeDtypeStruct((NK, DIM), jnp.float32),
            jax.ShapeDtypeStruct((8, DIM), jnp.float32),
        ],
    )(kj, qa, delta, wp8, bp, scp, offp, wat)


def _t5_body(spre_ref, vj_ref, delta_ref, ft_ref, wp8_ref, bp_ref, scp_ref,
             offp_ref, sca_ref, offa_ref, out_ref, st_ref):
    nq = _PT // K
    pe_pre = jnp.dot(delta_ref[...], wp8_ref[...], preferred_element_type=jnp.float32) + bp_ref[...]
    pe = jnp.maximum(pe_pre * scp_ref[...] + offp_ref[...], 0.0)
    attn = jnp.maximum(spre_ref[...] * sca_ref[...] + offa_ref[...], 0.0)
    s = jnp.sum(attn, axis=1, keepdims=True)
    s2 = s.reshape(nq, K)
    m = jnp.max(s2, axis=1, keepdims=True)
    e = jnp.exp(s2 - m)
    w2 = e / jnp.sum(e, axis=1, keepdims=True)
    w = w2.reshape(_PT, 1)
    agg = (vj_ref[...] + pe) * w
    outp = ft_ref[...] + jnp.sum(agg.reshape(nq, K, DIM), axis=1)
    out_ref[...] = outp
    ssum = jnp.sum(outp, axis=0, keepdims=True)
    ssq = jnp.sum(outp * outp, axis=0, keepdims=True)
    blk = jnp.concatenate([ssum, ssq, jnp.zeros((6, DIM), jnp.float32)], axis=0)
    i = pl.program_id(0)

    @pl.when(i == 0)
    def _init():
        st_ref[...] = blk

    @pl.when(i > 0)
    def _acc():
        st_ref[...] = st_ref[...] + blk


def _t5(spre, vj, delta, fT, wp8, bp, scp, offp, sca, offa):
    row = pl.BlockSpec((1, DIM), lambda i: (0, 0))
    return pl.pallas_call(
        _t5_body,
        grid=(NK // _PT,),
        in_specs=[
            pl.BlockSpec((_PT, DIM), lambda i: (i, 0)),
            pl.BlockSpec((_PT, DIM), lambda i: (i, 0)),
            pl.BlockSpec((_PT, 8), lambda i: (i, 0)),
            pl.BlockSpec((_PT // K, DIM), lambda i: (i, 0)),
            pl.BlockSpec((8, DIM), lambda i: (0, 0)),
            row, row, row, row, row,
        ],
        out_specs=[
            pl.BlockSpec((_PT // K, DIM), lambda i: (i, 0)),
            pl.BlockSpec((8, DIM), lambda i: (0, 0)),
        ],
        out_shape=[
            jax.ShapeDtypeStruct((N, DIM), jnp.float32),
            jax.ShapeDtypeStruct((8, DIM), jnp.float32),
        ],
    )(spre, vj, delta, fT, wp8, bp, scp, offp, sca, offa)


def _t6_body(o_ref, sco_ref, offo_ref, out_ref):
    out_ref[...] = jnp.maximum(o_ref[...] * sco_ref[...] + offo_ref[...], 0.0)


def _t6(outp, sco, offo):
    row = pl.BlockSpec((1, DIM), lambda i: (0, 0))
    blk = pl.BlockSpec((_NT, DIM), lambda i: (i, 0))
    return pl.pallas_call(
        _t6_body,
        grid=(N // _NT,),
        in_specs=[blk, row, row],
        out_specs=blk,
        out_shape=jax.ShapeDtypeStruct((N, DIM), jnp.float32),
    )(outp, sco, offo)


_SC_WORKERS = 32
_ROWS_W = N // _SC_WORKERS


def _s2_sc(selc, xr, yr, zr, xs, ys, zs, xxs):
    mesh = plsc.VectorSubcoreMesh(core_axis_name="c", subcore_axis_name="s")

    @functools.partial(
        pl.kernel,
        mesh=mesh,
        compiler_params=pltpu.CompilerParams(needs_layout_passes=False),
        out_type=[
            jax.ShapeDtypeStruct((NK,), jnp.int32),
            jax.ShapeDtypeStruct((NK * 8,), jnp.float32),
        ],
        scratch_types=[
            pltpu.VMEM((N,), jnp.float32),
            pltpu.VMEM((N,), jnp.float32),
            pltpu.VMEM((N,), jnp.float32),
            pltpu.VMEM((N,), jnp.float32),
            pltpu.VMEM((N,), jnp.float32),
            pltpu.VMEM((N,), jnp.float32),
            pltpu.VMEM((N,), jnp.float32),
            pltpu.VMEM((_ROWS_W * K,), jnp.int32),
            pltpu.VMEM((256,), jnp.float32),
            pltpu.VMEM((256,), jnp.int32),
            pltpu.VMEM((_ROWS_W * K,), jnp.int32),
            pltpu.VMEM((_ROWS_W * K * 8,), jnp.float32),
        ],
    )
    def knn(selc_h, xr_h, yr_h, zr_h, x_h, y_h, z_h, xx_h, idx_h, delta_h,
            xrt, yrt, zrt, xt, yt, zt, xxt, selv, cz, cidx, idxs, dels):
        wid = lax.axis_index("s") * 2 + lax.axis_index("c")
        base = wid * _ROWS_W
        pltpu.sync_copy(xr_h, xrt)
        pltpu.sync_copy(yr_h, yrt)
        pltpu.sync_copy(zr_h, zrt)
        pltpu.sync_copy(x_h, xt)
        pltpu.sync_copy(y_h, yt)
        pltpu.sync_copy(z_h, zt)
        pltpu.sync_copy(xx_h, xxt)
        pltpu.sync_copy(selc_h.at[pl.ds(base * K, _ROWS_W * K)], selv)
        lanes = lax.iota(jnp.int32, 16)

        def row_body(r, _):
            n = base + r
            nv = jnp.full((16,), 0, jnp.int32) + n
            xq = plsc.load_gather(xrt, [nv])
            yq = plsc.load_gather(yrt, [nv])
            zq = plsc.load_gather(zrt, [nv])
            xxq = plsc.load_gather(xxt, [nv])
            csel = selv[pl.ds(r * K, K)]
            for t in range(CHW):
                midx = csel + jnp.int32(NCHUNK * t)
                xm = plsc.load_gather(xrt, [midx])
                ym = plsc.load_gather(yrt, [midx])
                zm = plsc.load_gather(zrt, [midx])
                xxm = plsc.load_gather(xxt, [midx])
                zval = (xxq + xxm) - 2.0 * ((xq * xm + yq * ym) + zq * zm)
                cz[pl.ds(16 * t, 16)] = jnp.maximum(zval, 0.0)
                cidx[pl.ds(16 * t, 16)] = midx
            tk = jnp.full((16,), BIG, jnp.float32)
            ti = jnp.full((16,), 0, jnp.int32)
            for t in range(CHW):
                sk, si = plsc.sort_key_val(cz[pl.ds(16 * t, 16)],
                                           cidx[pl.ds(16 * t, 16)],
                                           descending=True)
                m = (tk < sk) | ((tk == sk) & (ti <= si))
                nk_ = jnp.where(m, tk, sk)
                ni_ = jnp.where(m, ti, si)
                tk, ti = plsc.sort_key_val(nk_, ni_)
            idxs[pl.ds(r * K, K)] = ti
            xqf = plsc.load_gather(xt, [nv])
            yqf = plsc.load_gather(yt, [nv])
            zqf = plsc.load_gather(zt, [nv])
            xn = plsc.load_gather(xt, [ti]) - xqf
            yn = plsc.load_gather(yt, [ti]) - yqf
            zn = plsc.load_gather(zt, [ti]) - zqf
            off = r * (K * 8) + lanes * 8
            plsc.store_scatter(dels, [off], xn)
            plsc.store_scatter(dels, [off + 1], yn)
            plsc.store_scatter(dels, [off + 2], zn)
            return ()

        zero = jnp.zeros((16,), jnp.float32)
        for b in range(_ROWS_W * K * 8 // 16):
            dels[pl.ds(16 * b, 16)] = zero
        lax.fori_loop(0, _ROWS_W, row_body, ())
        pltpu.sync_copy(idxs, idx_h.at[pl.ds(base * K, _ROWS_W * K)])
        pltpu.sync_copy(dels, delta_h.at[pl.ds(base * K * 8, _ROWS_W * K * 8)])

    return knn(selc, xr, yr, zr, xs, ys, zs, xxs)


_GCH = 128


def _s3_sc(idxf, ka, v):
    mesh = plsc.VectorSubcoreMesh(core_axis_name="c", subcore_axis_name="s")
    per_w = NK // _SC_WORKERS

    @functools.partial(
        pl.kernel,
        mesh=mesh,
        compiler_params=pltpu.CompilerParams(needs_layout_passes=False),
        out_type=[
            jax.ShapeDtypeStruct((NK, DIM), jnp.float32),
            jax.ShapeDtypeStruct((NK, DIM), jnp.float32),
        ],
        scratch_types=[
            pltpu.VMEM((per_w,), jnp.int32),
            pltpu.VMEM((_GCH, DIM), jnp.float32),
            pltpu.VMEM((_GCH, DIM), jnp.float32),
            pltpu.SemaphoreType.DMA,
            pltpu.SemaphoreType.DMA,
        ],
    )
    def gat(idx_h, ka_h, v_h, kj_h, vj_h, idxv, kbuf, vbuf, sem1, sem2):
        wid = lax.axis_index("s") * 2 + lax.axis_index("c")
        base = wid * per_w
        pltpu.sync_copy(idx_h.at[pl.ds(base, per_w)], idxv)

        def chunk_body(g, _):
            off = g * _GCH
            cp1 = pltpu.async_copy(ka_h.at[idxv.at[pl.ds(off, _GCH)]], kbuf, sem1)
            cp2 = pltpu.async_copy(v_h.at[idxv.at[pl.ds(off, _GCH)]], vbuf, sem2)
            cp1.wait()
            cp2.wait()
            pltpu.sync_copy(kbuf, kj_h.at[pl.ds(base + off, _GCH)])
            pltpu.sync_copy(vbuf, vj_h.at[pl.ds(base + off, _GCH)])
            return ()

        lax.fori_loop(0, per_w // _GCH, chunk_body, ())

    return gat(idxf, ka, v)


def _pool_stats(st, cnt):
    s = st[0].reshape(32, 4).sum(axis=1)
    ss = st[1].reshape(32, 4).sum(axis=1)
    mean = s / cnt
    var = ss / cnt - mean * mean
    inv = 1.0 / jnp.sqrt(var + 1e-5)
    return jnp.repeat(mean, 4), jnp.repeat(inv, 4)


def kernel(xyz_bcn, feat, Wq, bq, Wk, bk, Wv, bv, Wp, bp, gp, bgp, Wa, ba, ga,
           bga, gn_g, gn_b):
    xyz = xyz_bcn[0]
    xyzT = xyz.T
    xyzT8 = jnp.pad(xyzT, ((0, 0), (0, 5)))
    xyz8 = jnp.pad(xyz, ((0, 5), (0, 0)))
    fT = feat[0].T

    a1t = (Wa @ Wk).T
    a2t = (Wa @ Wq).T
    wvt = Wv.T
    bka = (Wa @ bk + ba)[None, :]
    bqa = (Wa @ bq)[None, :]
    bv2 = bv[None, :]
    wp8 = jnp.pad(Wp.T, ((0, 5), (0, 0)))
    bp2 = bp[None, :]

    cmt, xx2, xyzr8 = _t0(xyzT8, xyz8)
    ka, qa, v = _t0b(fT, a1t, a2t, wvt, bka, bqa, bv2)
    selc = _t1(cmt)

    xs, ys, zs = xyzT[:, 0], xyzT[:, 1], xyzT[:, 2]
    xr, yr, zr = xyzr8[:, 0], xyzr8[:, 1], xyzr8[:, 2]
    idxf, delta_flat = _s2_sc(selc.reshape(NK), xr, yr, zr, xs, ys, zs,
                              xx2[:, 0])
    delta = delta_flat.reshape(NK, 8)

    kj, vj = _s3_sc(idxf, ka, v)

    pst = _t3b(delta, wp8, bp2)
    pmean, pinv = _pool_stats(pst, float(NK * 4))
    scp = (pinv * gp)[None, :]
    offp = (bgp - pmean * pinv * gp)[None, :]

    spre, ast = _t4(kj, qa, delta, wp8, bp2, scp, offp, Wa.T)
    amean, ainv = _pool_stats(ast, float(NK * 4))
    sca = (ainv * ga)[None, :]
    offa = (bga - amean * ainv * ga)[None, :]

    outp, ost = _t5(spre, vj, delta, fT, wp8, bp2, scp, offp, sca, offa)
    omean, oinv = _pool_stats(ost, float(N * 4))
    sco = (oinv * gn_g)[None, :]
    offo = (gn_b - omean * oinv * gn_g)[None, :]

    out = _t6(outp, sco, offo)
    return out.T[None]
